# Initial kernel scaffold; baseline (speedup 1.0000x reference)
import jax
import jax.numpy as jnp
from jax.experimental import pallas as pl


def kernel(x, edge_index, W1, b1, W2, b2, W_out, b_out):
    raise NotImplementedError("write your pallas kernel here")


# speedup vs baseline: 30.9673x; 30.9673x over previous
import functools

import jax
import jax.numpy as jnp
from jax import lax
from jax.experimental import pallas as pl
from jax.experimental.pallas import tpu as pltpu
from jax.experimental.pallas import tpu_sc as plsc

N = 10000
D = 128
E = 320000
NCLS = 40
NPAD = 10240
NC = 2
NS = 16
NW = NC * NS
EPT = E // NW
C = 128
NCHUNK = 80
EPTP = NCHUNK * C
RPS = NPAD // NS
BM = 1024


def _vector_mesh():
    return plsc.VectorSubcoreMesh(core_axis_name="core", subcore_axis_name="subcore")


def _sc_degree(dstp, zeros16):
    @functools.partial(
        pl.kernel,
        out_type=jax.ShapeDtypeStruct((NC, NPAD, 16), jnp.float32),
        mesh=_vector_mesh(),
        scratch_types=[
            pltpu.VMEM((NCHUNK, C), jnp.int32),
            pltpu.VMEM((C, 16), jnp.float32),
            pltpu.VMEM_SHARED((NPAD, 16), jnp.float32),
            pltpu.SemaphoreType.DMA,
        ],
    )
    def k(dstp_hbm, z_hbm, out_hbm, dst_v, ones_v, acc_sp, sem):
        cid = lax.axis_index("core")
        sid = lax.axis_index("subcore")
        wid = cid * NS + sid

        pltpu.sync_copy(z_hbm.at[pl.ds(sid * RPS, RPS)],
                        acc_sp.at[pl.ds(sid * RPS, RPS)])
        pltpu.sync_copy(dstp_hbm.at[wid], dst_v)

        @pl.loop(0, C)
        def _(i):
            ones_v[i] = jnp.ones((16,), jnp.float32)

        plsc.subcore_barrier()

        NCH8 = NCHUNK - NCHUNK % 8

        @pl.loop(0, NCH8, step=8)
        def _(c):
            descs = [
                pltpu.async_copy(ones_v, acc_sp.at[dst_v.at[c + j]], sem, add=True)
                for j in range(8)
            ]
            for d_ in descs:
                d_.wait()

        tail = [
            pltpu.async_copy(ones_v, acc_sp.at[dst_v.at[c]], sem, add=True)
            for c in range(NCH8, NCHUNK)
        ]
        for d_ in tail:
            d_.wait()

        plsc.subcore_barrier()
        pltpu.sync_copy(acc_sp.at[pl.ds(sid * RPS, RPS)],
                        out_hbm.at[cid, pl.ds(sid * RPS, RPS)])

    return k(dstp, zeros16)


def _sc_scatter(hs, zeros_big, pck):
    @functools.partial(
        pl.kernel,
        out_type=jax.ShapeDtypeStruct((NC, NPAD, D), jnp.float32),
        mesh=_vector_mesh(),
        scratch_types=[
            pltpu.VMEM((NCHUNK, C), jnp.int32),
            pltpu.VMEM((1, C), jnp.int32),
            pltpu.VMEM((1, C), jnp.int32),
            pltpu.VMEM((1, C), jnp.int32),
            pltpu.VMEM((1, C), jnp.int32),
            pltpu.VMEM((C, D), jnp.float32),
            pltpu.VMEM((C, D), jnp.float32),
            pltpu.VMEM_SHARED((NPAD, D), jnp.float32),
            pltpu.SemaphoreType.DMA,
            pltpu.SemaphoreType.DMA,
        ],
    )
    def k(hs_hbm, z_hbm, pck_hbm, out_hbm,
          pck_v, src0, src1, dst0, dst1, buf0, buf1, acc_sp, sem0, sem1):
        cid = lax.axis_index("core")
        sid = lax.axis_index("subcore")
        wid = cid * NS + sid

        def unpack_src(c, row):
            for j in range(C // 16):
                v = pck_v[c, pl.ds(j * 16, 16)]
                row[0, pl.ds(j * 16, 16)] = lax.bitwise_and(v, 0xFFFF)

        def unpack_dst(c, row):
            for j in range(C // 16):
                v = pck_v[c, pl.ds(j * 16, 16)]
                row[0, pl.ds(j * 16, 16)] = lax.shift_right_logical(v, 16)

        @pl.when(cid == 0)
        def _():
            pltpu.sync_copy(hs_hbm.at[pl.ds(sid * RPS, RPS)],
                            acc_sp.at[pl.ds(sid * RPS, RPS)])

        @pl.when(cid != 0)
        def _():
            pltpu.sync_copy(z_hbm.at[pl.ds(sid * RPS, RPS)],
                            acc_sp.at[pl.ds(sid * RPS, RPS)])

        pltpu.sync_copy(pck_hbm.at[wid], pck_v)
        plsc.subcore_barrier()

        unpack_src(0, src0)
        pltpu.async_copy(hs_hbm.at[src0.at[0]], buf0, sem0)
        unpack_src(1, src1)
        pltpu.async_copy(hs_hbm.at[src1.at[0]], buf1, sem1)

        @pl.loop(0, NCHUNK - 2, step=2)
        def _(c):
            pltpu.make_async_copy(hs_hbm.at[src0.at[0]], buf0, sem0).wait()
            unpack_dst(c, dst0)
            pltpu.sync_copy(buf0, acc_sp.at[dst0.at[0]], add=True)
            unpack_src(c + 2, src0)
            pltpu.async_copy(hs_hbm.at[src0.at[0]], buf0, sem0)
            pltpu.make_async_copy(hs_hbm.at[src1.at[0]], buf1, sem1).wait()
            unpack_dst(c + 1, dst1)
            pltpu.sync_copy(buf1, acc_sp.at[dst1.at[0]], add=True)
            unpack_src(c + 3, src1)
            pltpu.async_copy(hs_hbm.at[src1.at[0]], buf1, sem1)

        pltpu.make_async_copy(hs_hbm.at[src0.at[0]], buf0, sem0).wait()
        unpack_dst(NCHUNK - 2, dst0)
        pltpu.sync_copy(buf0, acc_sp.at[dst0.at[0]], add=True)
        pltpu.make_async_copy(hs_hbm.at[src1.at[0]], buf1, sem1).wait()
        unpack_dst(NCHUNK - 1, dst1)
        pltpu.sync_copy(buf1, acc_sp.at[dst1.at[0]], add=True)

        plsc.subcore_barrier()
        pltpu.sync_copy(acc_sp.at[pl.ds(sid * RPS, RPS)],
                        out_hbm.at[cid, pl.ds(sid * RPS, RPS)])

    return k(hs, zeros_big, pck)


def _tc_scale_in(x_p, W1, degp):
    def body(x_ref, w_ref, dp_ref, hs_ref, dis_ref):
        d = dp_ref[0] + dp_ref[1]
        deg = d[:, 0:1] + 1.0
        dis = lax.rsqrt(deg)
        h = jnp.dot(x_ref[...], w_ref[...], preferred_element_type=jnp.float32)
        hs_ref[...] = h * dis
        dis_ref[...] = dis

    return pl.pallas_call(
        body,
        grid=(NPAD // BM,),
        in_specs=[
            pl.BlockSpec((BM, D), lambda i: (i, 0)),
            pl.BlockSpec((D, D), lambda i: (0, 0)),
            pl.BlockSpec((2, BM, 16), lambda i: (0, i, 0)),
        ],
        out_specs=[
            pl.BlockSpec((BM, D), lambda i: (i, 0)),
            pl.BlockSpec((BM, 1), lambda i: (i, 0)),
        ],
        out_shape=[
            jax.ShapeDtypeStruct((NPAD, D), jnp.float32),
            jax.ShapeDtypeStruct((NPAD, 1), jnp.float32),
        ],
    )(x_p, W1, degp)


def _tc_mid(P, dis, b1r, W2):
    def body(p_ref, dis_ref, b_ref, w_ref, hs2_ref):
        s = (p_ref[0] + p_ref[1]) * dis_ref[...] + b_ref[...]
        a = jnp.maximum(s, 0.0)
        h2 = jnp.dot(a, w_ref[...], preferred_element_type=jnp.float32)
        hs2_ref[...] = h2 * dis_ref[...]

    return pl.pallas_call(
        body,
        grid=(NPAD // BM,),
        in_specs=[
            pl.BlockSpec((2, BM, D), lambda i: (0, i, 0)),
            pl.BlockSpec((BM, 1), lambda i: (i, 0)),
            pl.BlockSpec((1, D), lambda i: (0, 0)),
            pl.BlockSpec((D, D), lambda i: (0, 0)),
        ],
        out_specs=pl.BlockSpec((BM, D), lambda i: (i, 0)),
        out_shape=jax.ShapeDtypeStruct((NPAD, D), jnp.float32),
    )(P, dis, b1r, W2)


def _tc_out(P, dis, b2r, Wo_p, bo_p):
    def body(p_ref, dis_ref, b_ref, wo_ref, bo_ref, o_ref):
        s = (p_ref[0] + p_ref[1]) * dis_ref[...] + b_ref[...]
        a = jnp.maximum(s, 0.0)
        logits = jnp.dot(a, wo_ref[...], preferred_element_type=jnp.float32)
        logits = logits + bo_ref[...]
        m = jnp.max(logits, axis=1, keepdims=True)
        e = jnp.exp(logits - m)
        o_ref[...] = e / jnp.sum(e, axis=1, keepdims=True)

    return pl.pallas_call(
        body,
        grid=(NPAD // BM,),
        in_specs=[
            pl.BlockSpec((2, BM, D), lambda i: (0, i, 0)),
            pl.BlockSpec((BM, 1), lambda i: (i, 0)),
            pl.BlockSpec((1, D), lambda i: (0, 0)),
            pl.BlockSpec((D, D), lambda i: (0, 0)),
            pl.BlockSpec((1, D), lambda i: (0, 0)),
        ],
        out_specs=pl.BlockSpec((BM, D), lambda i: (i, 0)),
        out_shape=jax.ShapeDtypeStruct((NPAD, D), jnp.float32),
    )(P, dis, b2r, Wo_p, bo_p)


def kernel(x, edge_index, W1, b1, W2, b2, W_out, b_out):
    f32 = jnp.float32
    src = edge_index[0].reshape(NW, EPT)
    dst = edge_index[1].reshape(NW, EPT)
    npadE = EPTP - EPT
    pad_src = jnp.broadcast_to((jnp.arange(npadE, dtype=jnp.int32) * 41) % N,
                               (NW, npadE))
    pad_dst = jnp.broadcast_to(N + jnp.arange(npadE, dtype=jnp.int32),
                               (NW, npadE))
    srcp = jnp.concatenate([src, pad_src], axis=1).reshape(NW, NCHUNK, C)
    dstp = jnp.concatenate([dst, pad_dst], axis=1).reshape(NW, NCHUNK, C)
    pck = (dstp << 16) | srcp

    x_p = jnp.zeros((NPAD, D), f32).at[:N].set(x)
    zeros_big = jnp.zeros((NPAD, D), f32)
    zeros16 = jnp.zeros((NPAD, 16), f32)
    b1r = b1.reshape(1, D)
    b2r = b2.reshape(1, D)
    Wo_p = jnp.zeros((D, D), f32).at[:, :NCLS].set(W_out)
    bo_p = jnp.full((1, D), -1e9, f32).at[0, :NCLS].set(b_out)

    degp = _sc_degree(dstp, zeros16)
    hs1, dis = _tc_scale_in(x_p, W1, degp)
    P1 = _sc_scatter(hs1, zeros_big, pck)
    hs2 = _tc_mid(P1, dis, b1r, W2)
    P2 = _sc_scatter(hs2, zeros_big, pck)
    out = _tc_out(P2, dis, b2r, Wo_p, bo_p)
    return out[:N, :NCLS]


# speedup vs baseline: 31.0561x; 1.0029x over previous
import functools

import jax
import jax.numpy as jnp
from jax import lax
from jax.experimental import pallas as pl
from jax.experimental.pallas import tpu as pltpu
from jax.experimental.pallas import tpu_sc as plsc

N = 10000
D = 128
E = 320000
NCLS = 40
NPAD = 10240
NC = 2
NS = 16
NW = NC * NS
EPT = E // NW
C = 128
NCHUNK = 80
EPTP = NCHUNK * C
RPS = NPAD // NS
BM = 1024


def _vector_mesh():
    return plsc.VectorSubcoreMesh(core_axis_name="core", subcore_axis_name="subcore")


def _sc_degree(dstp, zeros16):
    @functools.partial(
        pl.kernel,
        out_type=jax.ShapeDtypeStruct((NC, NPAD, 16), jnp.float32),
        mesh=_vector_mesh(),
        scratch_types=[
            pltpu.VMEM((NCHUNK, C), jnp.int32),
            pltpu.VMEM((C, 16), jnp.float32),
            pltpu.VMEM_SHARED((NPAD, 16), jnp.float32),
            pltpu.SemaphoreType.DMA,
        ],
    )
    def k(dstp_hbm, z_hbm, out_hbm, dst_v, ones_v, acc_sp, sem):
        cid = lax.axis_index("core")
        sid = lax.axis_index("subcore")
        wid = cid * NS + sid

        pltpu.sync_copy(z_hbm.at[pl.ds(sid * RPS, RPS)],
                        acc_sp.at[pl.ds(sid * RPS, RPS)])
        pltpu.sync_copy(dstp_hbm.at[wid], dst_v)

        @pl.loop(0, C)
        def _(i):
            ones_v[i] = jnp.ones((16,), jnp.float32)

        plsc.subcore_barrier()

        NCH8 = NCHUNK - NCHUNK % 8

        @pl.loop(0, NCH8, step=8)
        def _(c):
            descs = [
                pltpu.async_copy(ones_v, acc_sp.at[dst_v.at[c + j]], sem, add=True)
                for j in range(8)
            ]
            for d_ in descs:
                d_.wait()

        tail = [
            pltpu.async_copy(ones_v, acc_sp.at[dst_v.at[c]], sem, add=True)
            for c in range(NCH8, NCHUNK)
        ]
        for d_ in tail:
            d_.wait()

        plsc.subcore_barrier()
        pltpu.sync_copy(acc_sp.at[pl.ds(sid * RPS, RPS)],
                        out_hbm.at[cid, pl.ds(sid * RPS, RPS)])

    return k(dstp, zeros16)


def _sc_scatter(hs, zeros_big, pck):
    @functools.partial(
        pl.kernel,
        out_type=jax.ShapeDtypeStruct((NC, NPAD, D), jnp.float32),
        mesh=_vector_mesh(),
        scratch_types=[
            pltpu.VMEM((NCHUNK, C), jnp.int32),
            pltpu.VMEM((1, C), jnp.int32),
            pltpu.VMEM((1, C), jnp.int32),
            pltpu.VMEM((1, C), jnp.int32),
            pltpu.VMEM((1, C), jnp.int32),
            pltpu.VMEM((C, D), jnp.float32),
            pltpu.VMEM((C, D), jnp.float32),
            pltpu.VMEM_SHARED((NPAD, D), jnp.float32),
            pltpu.SemaphoreType.DMA,
            pltpu.SemaphoreType.DMA,
        ],
    )
    def k(hs_hbm, z_hbm, pck_hbm, out_hbm,
          pck_v, src0, src1, dst0, dst1, buf0, buf1, acc_sp, sem0, sem1):
        cid = lax.axis_index("core")
        sid = lax.axis_index("subcore")
        wid = cid * NS + sid

        def unpack_src(c, row):
            for j in range(C // 16):
                v = pck_v[c, pl.ds(j * 16, 16)]
                row[0, pl.ds(j * 16, 16)] = lax.bitwise_and(v, 0xFFFF)

        def unpack_dst(c, row):
            for j in range(C // 16):
                v = pck_v[c, pl.ds(j * 16, 16)]
                row[0, pl.ds(j * 16, 16)] = lax.shift_right_logical(v, 16)

        @pl.when(cid == 0)
        def _():
            pltpu.sync_copy(hs_hbm.at[pl.ds(sid * RPS, RPS)],
                            acc_sp.at[pl.ds(sid * RPS, RPS)])

        @pl.when(cid != 0)
        def _():
            pltpu.sync_copy(z_hbm.at[pl.ds(sid * RPS, RPS)],
                            acc_sp.at[pl.ds(sid * RPS, RPS)])

        pltpu.sync_copy(pck_hbm.at[wid], pck_v)
        plsc.subcore_barrier()

        unpack_src(0, src0)
        pltpu.async_copy(hs_hbm.at[src0.at[0]], buf0, sem0)
        unpack_src(1, src1)
        pltpu.async_copy(hs_hbm.at[src1.at[0]], buf1, sem1)

        @pl.loop(0, NCHUNK - 2, step=2)
        def _(c):
            pltpu.make_async_copy(hs_hbm.at[src0.at[0]], buf0, sem0).wait()
            unpack_dst(c, dst0)
            pltpu.sync_copy(buf0, acc_sp.at[dst0.at[0]], add=True)
            unpack_src(c + 2, src0)
            pltpu.async_copy(hs_hbm.at[src0.at[0]], buf0, sem0)
            pltpu.make_async_copy(hs_hbm.at[src1.at[0]], buf1, sem1).wait()
            unpack_dst(c + 1, dst1)
            pltpu.sync_copy(buf1, acc_sp.at[dst1.at[0]], add=True)
            unpack_src(c + 3, src1)
            pltpu.async_copy(hs_hbm.at[src1.at[0]], buf1, sem1)

        pltpu.make_async_copy(hs_hbm.at[src0.at[0]], buf0, sem0).wait()
        unpack_dst(NCHUNK - 2, dst0)
        pltpu.sync_copy(buf0, acc_sp.at[dst0.at[0]], add=True)
        pltpu.make_async_copy(hs_hbm.at[src1.at[0]], buf1, sem1).wait()
        unpack_dst(NCHUNK - 1, dst1)
        pltpu.sync_copy(buf1, acc_sp.at[dst1.at[0]], add=True)

        plsc.subcore_barrier()
        pltpu.sync_copy(acc_sp.at[pl.ds(sid * RPS, RPS)],
                        out_hbm.at[cid, pl.ds(sid * RPS, RPS)])

    return k(hs, zeros_big, pck)


def _tc_mm1(x_p, W1):
    def body(x_ref, w_ref, h_ref):
        h_ref[...] = jnp.dot(x_ref[...], w_ref[...],
                             preferred_element_type=jnp.float32)

    return pl.pallas_call(
        body,
        grid=(NPAD // BM,),
        in_specs=[
            pl.BlockSpec((BM, D), lambda i: (i, 0)),
            pl.BlockSpec((D, D), lambda i: (0, 0)),
        ],
        out_specs=pl.BlockSpec((BM, D), lambda i: (i, 0)),
        out_shape=jax.ShapeDtypeStruct((NPAD, D), jnp.float32),
    )(x_p, W1)


def _tc_scale(h, degp):
    def body(h_ref, dp_ref, hs_ref, dis_ref):
        d = dp_ref[0] + dp_ref[1]
        deg = d[:, 0:1] + 1.0
        dis = lax.rsqrt(deg)
        hs_ref[...] = h_ref[...] * dis
        dis_ref[...] = dis

    return pl.pallas_call(
        body,
        grid=(NPAD // BM,),
        in_specs=[
            pl.BlockSpec((BM, D), lambda i: (i, 0)),
            pl.BlockSpec((2, BM, 16), lambda i: (0, i, 0)),
        ],
        out_specs=[
            pl.BlockSpec((BM, D), lambda i: (i, 0)),
            pl.BlockSpec((BM, 1), lambda i: (i, 0)),
        ],
        out_shape=[
            jax.ShapeDtypeStruct((NPAD, D), jnp.float32),
            jax.ShapeDtypeStruct((NPAD, 1), jnp.float32),
        ],
    )(h, degp)


def _tc_mid(P, dis, b1r, W2):
    def body(p_ref, dis_ref, b_ref, w_ref, hs2_ref):
        s = (p_ref[0] + p_ref[1]) * dis_ref[...] + b_ref[...]
        a = jnp.maximum(s, 0.0)
        h2 = jnp.dot(a, w_ref[...], preferred_element_type=jnp.float32)
        hs2_ref[...] = h2 * dis_ref[...]

    return pl.pallas_call(
        body,
        grid=(NPAD // BM,),
        in_specs=[
            pl.BlockSpec((2, BM, D), lambda i: (0, i, 0)),
            pl.BlockSpec((BM, 1), lambda i: (i, 0)),
            pl.BlockSpec((1, D), lambda i: (0, 0)),
            pl.BlockSpec((D, D), lambda i: (0, 0)),
        ],
        out_specs=pl.BlockSpec((BM, D), lambda i: (i, 0)),
        out_shape=jax.ShapeDtypeStruct((NPAD, D), jnp.float32),
    )(P, dis, b1r, W2)


def _tc_out(P, dis, b2r, Wo_p, bo_p):
    def body(p_ref, dis_ref, b_ref, wo_ref, bo_ref, o_ref):
        s = (p_ref[0] + p_ref[1]) * dis_ref[...] + b_ref[...]
        a = jnp.maximum(s, 0.0)
        logits = jnp.dot(a, wo_ref[...], preferred_element_type=jnp.float32)
        logits = logits + bo_ref[...]
        m = jnp.max(logits, axis=1, keepdims=True)
        e = jnp.exp(logits - m)
        o_ref[...] = e / jnp.sum(e, axis=1, keepdims=True)

    return pl.pallas_call(
        body,
        grid=(NPAD // BM,),
        in_specs=[
            pl.BlockSpec((2, BM, D), lambda i: (0, i, 0)),
            pl.BlockSpec((BM, 1), lambda i: (i, 0)),
            pl.BlockSpec((1, D), lambda i: (0, 0)),
            pl.BlockSpec((D, D), lambda i: (0, 0)),
            pl.BlockSpec((1, D), lambda i: (0, 0)),
        ],
        out_specs=pl.BlockSpec((BM, D), lambda i: (i, 0)),
        out_shape=jax.ShapeDtypeStruct((NPAD, D), jnp.float32),
    )(P, dis, b2r, Wo_p, bo_p)


def kernel(x, edge_index, W1, b1, W2, b2, W_out, b_out):
    f32 = jnp.float32
    src = edge_index[0].reshape(NW, EPT)
    dst = edge_index[1].reshape(NW, EPT)
    npadE = EPTP - EPT
    pad_src = jnp.broadcast_to((jnp.arange(npadE, dtype=jnp.int32) * 41) % N,
                               (NW, npadE))
    pad_dst = jnp.broadcast_to(N + jnp.arange(npadE, dtype=jnp.int32),
                               (NW, npadE))
    srcp = jnp.concatenate([src, pad_src], axis=1).reshape(NW, NCHUNK, C)
    dstp = jnp.concatenate([dst, pad_dst], axis=1).reshape(NW, NCHUNK, C)
    pck = (dstp << 16) | srcp

    x_p = jnp.zeros((NPAD, D), f32).at[:N].set(x)
    zeros_big = jnp.zeros((NPAD, D), f32)
    zeros16 = jnp.zeros((NPAD, 16), f32)
    b1r = b1.reshape(1, D)
    b2r = b2.reshape(1, D)
    Wo_p = jnp.zeros((D, D), f32).at[:, :NCLS].set(W_out)
    bo_p = jnp.full((1, D), -1e9, f32).at[0, :NCLS].set(b_out)

    degp = _sc_degree(dstp, zeros16)
    h1 = _tc_mm1(x_p, W1)
    hs1, dis = _tc_scale(h1, degp)
    P1 = _sc_scatter(hs1, zeros_big, pck)
    hs2 = _tc_mid(P1, dis, b1r, W2)
    P2 = _sc_scatter(hs2, zeros_big, pck)
    out = _tc_out(P2, dis, b2r, Wo_p, bo_p)
    return out[:N, :NCLS]
